# baseline (speedup 1.0000x reference)
import functools

import jax
import jax.numpy as jnp
from jax.experimental import pallas as pl
from jax.experimental.pallas import tpu as pltpu
from jax.experimental.pallas import tpu_sc as plsc

_HEADS = 16
_NUM_BUCKETS = 32
_SCALE = 0.125
_THRESHOLDS = (19, 23, 42, 218)
_MAX_BUCKET = 19

_BI = 256
_SEQ = 2048
_WW = _BI + _SEQ
_LW = 4096
_LINE_PAD = 16


def _sc_lsw_kernel(table_hbm, lsw_hbm, idx_v, line_v, row_v, sem):
    wid = jax.lax.axis_index("c") * 16 + jax.lax.axis_index("s")
    h = wid // 2
    s0 = 4 * (wid % 2)

    lane = jax.lax.broadcasted_iota(jnp.int32, (16,), 0)
    base = 2043 + s0
    h_vec = jnp.broadcast_to(h, (16,))
    zero = jnp.zeros((16,), jnp.int32)
    one = jnp.full((16,), 1, jnp.int32)

    def idx_body(c, carry):
        n = jnp.maximum(jnp.broadcast_to(base - c * 16, (16,)) - lane, zero)
        large = jnp.full((16,), 15, jnp.int32)
        for t in _THRESHOLDS:
            large = large + jnp.where(n >= jnp.full((16,), t, jnp.int32), one, zero)
        bucket = jnp.where(n < jnp.full((16,), 16, jnp.int32), n,
                           jnp.where(n == jnp.full((16,), 16, jnp.int32), zero, large))
        idx_v[pl.ds(c * 16, 16)] = bucket * _HEADS + h_vec
        return carry

    jax.lax.fori_loop(0, (_LW + _LINE_PAD) // 16, idx_body, 0)

    pltpu.async_copy(table_hbm.at[idx_v], line_v, sem).wait()

    for s_rel in range(4):
        off = 3 - s_rel

        def shear_body(c, carry):
            row_v[pl.ds(c * 16, 16)] = line_v[pl.ds(c * 16 + off, 16)] * _SCALE
            return carry

        jax.lax.fori_loop(0, _LW // 16, shear_body, 0)
        pltpu.sync_copy(row_v, lsw_hbm.at[h, s0 + s_rel])


def _sc_lsw(rel_bias_table):
    table_flat = rel_bias_table.reshape(_NUM_BUCKETS * _HEADS)
    run = functools.partial(
        pl.kernel,
        out_type=jax.ShapeDtypeStruct((_HEADS, 8, _LW), jnp.float32),
        mesh=plsc.VectorSubcoreMesh(
            core_axis_name="c", subcore_axis_name="s",
            num_cores=2, num_subcores=16),
        scratch_types=[
            pltpu.VMEM((_LW + _LINE_PAD,), jnp.int32),
            pltpu.VMEM((_LW + _LINE_PAD,), jnp.float32),
            pltpu.VMEM((_LW,), jnp.float32),
            pltpu.SemaphoreType.DMA,
        ],
    )(_sc_lsw_kernel)
    return run(table_flat)


def _bias_add_kernel(lsw_ref, qk_ref, out_ref):
    ib = pl.program_id(1)
    n_ib = pl.num_programs(1)

    w = lsw_ref[0, :, pl.ds(_BI * (n_ib - 1 - ib), _WW)]

    for g in range(_BI // 8):
        off = (_BI - 8) - 8 * g
        r = 8 * g
        out_ref[0, 0, r:r + 8, :] = (
            qk_ref[0, 0, r:r + 8, :] + w[:, off:off + _SEQ])


def kernel(qk_dots, rel_bias_table):
    lsw = _sc_lsw(rel_bias_table)
    n_ib = _SEQ // _BI
    return pl.pallas_call(
        _bias_add_kernel,
        grid=(_HEADS, n_ib),
        in_specs=[
            pl.BlockSpec((1, 8, _LW), lambda h, ib: (h, 0, 0)),
            pl.BlockSpec((1, 1, _BI, _SEQ), lambda h, ib: (0, h, ib, 0)),
        ],
        out_specs=pl.BlockSpec((1, 1, _BI, _SEQ), lambda h, ib: (0, h, ib, 0)),
        out_shape=jax.ShapeDtypeStruct(qk_dots.shape, qk_dots.dtype),
        compiler_params=pltpu.CompilerParams(
            dimension_semantics=("parallel", "parallel")),
    )(lsw, qk_dots)


# speedup vs baseline: 3.7010x; 3.7010x over previous
import functools

import jax
import jax.numpy as jnp
from jax.experimental import pallas as pl
from jax.experimental.pallas import tpu as pltpu
from jax.experimental.pallas import tpu_sc as plsc

_HEADS = 16
_NUM_BUCKETS = 32
_SCALE = 0.125
_THRESHOLDS = (19, 23, 42, 218)
_MAX_BUCKET = 19

_BI = 256
_SEQ = 2048
_WW = _BI + _SEQ
_LW = 4096
_LINE_PAD = 16


def _sc_lsw_kernel(table_hbm, lsw_hbm, tcol_v, line_v, row_v):
    wid = jax.lax.axis_index("c") * 16 + jax.lax.axis_index("s")
    h = wid // 2
    s0 = 4 * (wid % 2)

    pltpu.sync_copy(table_hbm.at[h], tcol_v)
    t_lo = tcol_v[pl.ds(0, 16)] * _SCALE
    t_hi = tcol_v[pl.ds(16, 16)] * _SCALE

    lane = jax.lax.broadcasted_iota(jnp.int32, (16,), 0)
    base = 2043 + s0
    zero = jnp.zeros((16,), jnp.int32)
    one = jnp.full((16,), 1, jnp.int32)
    k16 = jnp.full((16,), 16, jnp.int32)

    def line_body(c, carry):
        for u in range(4):
            m0 = (c * 4 + u) * 16
            n = jnp.maximum(jnp.broadcast_to(base - m0, (16,)) - lane, zero)
            large = jnp.full((16,), 15, jnp.int32)
            for t in _THRESHOLDS:
                large = large + jnp.where(n >= jnp.full((16,), t, jnp.int32),
                                          one, zero)
            bucket = jnp.where(n < k16, n, jnp.where(n == k16, zero, large))
            v_lo = t_lo.at[jnp.minimum(bucket, 15)].get(
                mode="promise_in_bounds")
            v_hi = t_hi.at[jnp.maximum(bucket - 16, 0)].get(
                mode="promise_in_bounds")
            line_v[pl.ds(m0, 16)] = jnp.where(bucket < k16, v_lo, v_hi)
        return carry

    jax.lax.fori_loop(0, (_LW + _LINE_PAD) // 64, line_body, 0)

    for s_rel in range(4):
        off = 3 - s_rel

        def shear_body(c, carry):
            for u in range(4):
                m0 = (c * 4 + u) * 16
                row_v[pl.ds(m0, 16)] = line_v[pl.ds(m0 + off, 16)]
            return carry

        jax.lax.fori_loop(0, _LW // 64, shear_body, 0)
        pltpu.sync_copy(row_v, lsw_hbm.at[h, s0 + s_rel])


def _sc_lsw(rel_bias_table):
    table_t = jnp.transpose(rel_bias_table)
    run = functools.partial(
        pl.kernel,
        out_type=jax.ShapeDtypeStruct((_HEADS, 8, _LW), jnp.float32),
        mesh=plsc.VectorSubcoreMesh(
            core_axis_name="c", subcore_axis_name="s",
            num_cores=2, num_subcores=16),
        scratch_types=[
            pltpu.VMEM((_NUM_BUCKETS,), jnp.float32),
            pltpu.VMEM((_LW + _LINE_PAD,), jnp.float32),
            pltpu.VMEM((_LW,), jnp.float32),
        ],
    )(_sc_lsw_kernel)
    return run(table_t)


def _bias_add_kernel(lsw_ref, qk_ref, out_ref):
    ib = pl.program_id(1)
    n_ib = pl.num_programs(1)

    w = lsw_ref[0, :, pl.ds(_BI * (n_ib - 1 - ib), _WW)]

    for g in range(_BI // 8):
        off = (_BI - 8) - 8 * g
        r = 8 * g
        out_ref[0, 0, r:r + 8, :] = (
            qk_ref[0, 0, r:r + 8, :] + w[:, off:off + _SEQ])


def kernel(qk_dots, rel_bias_table):
    lsw = _sc_lsw(rel_bias_table)
    n_ib = _SEQ // _BI
    return pl.pallas_call(
        _bias_add_kernel,
        grid=(_HEADS, n_ib),
        in_specs=[
            pl.BlockSpec((1, 8, _LW), lambda h, ib: (h, 0, 0)),
            pl.BlockSpec((1, 1, _BI, _SEQ), lambda h, ib: (0, h, ib, 0)),
        ],
        out_specs=pl.BlockSpec((1, 1, _BI, _SEQ), lambda h, ib: (0, h, ib, 0)),
        out_shape=jax.ShapeDtypeStruct(qk_dots.shape, qk_dots.dtype),
        compiler_params=pltpu.CompilerParams(
            dimension_semantics=("parallel", "parallel")),
    )(lsw, qk_dots)


# speedup vs baseline: 3.7607x; 1.0162x over previous
import functools

import jax
import jax.numpy as jnp
from jax.experimental import pallas as pl
from jax.experimental.pallas import tpu as pltpu
from jax.experimental.pallas import tpu_sc as plsc

_HEADS = 16
_NUM_BUCKETS = 32
_SCALE = 0.125
_THRESHOLDS = (19, 23, 42, 218)
_MAX_BUCKET = 19

_BI = 256
_SEQ = 2048
_WW = _BI + _SEQ
_LW = 4096
_LINE_PAD = 16


def _sc_lsw_kernel(table_hbm, lsw_hbm, tcol_v, line_v, row_v):
    wid = jax.lax.axis_index("c") * 16 + jax.lax.axis_index("s")
    h = wid // 2
    s0 = 4 * (wid % 2)

    pltpu.sync_copy(table_hbm.at[h], tcol_v)
    t_lo = tcol_v[pl.ds(0, 16)] * _SCALE
    t_hi = tcol_v[pl.ds(16, 16)] * _SCALE

    lane = jax.lax.broadcasted_iota(jnp.int32, (16,), 0)
    base = 2043 + s0
    zero = jnp.zeros((16,), jnp.int32)
    one = jnp.full((16,), 1, jnp.int32)
    k16 = jnp.full((16,), 16, jnp.int32)

    def line_body(c, carry):
        for u in range(4):
            m0 = (c * 4 + u) * 16
            n = jnp.maximum(jnp.broadcast_to(base - m0, (16,)) - lane, zero)
            large = jnp.full((16,), 15, jnp.int32)
            for t in _THRESHOLDS:
                large = large + jnp.where(n >= jnp.full((16,), t, jnp.int32),
                                          one, zero)
            bucket = jnp.where(n < k16, n, jnp.where(n == k16, zero, large))
            v_lo = t_lo.at[jnp.minimum(bucket, 15)].get(
                mode="promise_in_bounds")
            v_hi = t_hi.at[jnp.maximum(bucket - 16, 0)].get(
                mode="promise_in_bounds")
            line_v[pl.ds(m0, 16)] = jnp.where(bucket < k16, v_lo, v_hi)
        return carry

    jax.lax.fori_loop(0, (_LW + _LINE_PAD) // 64, line_body, 0)

    for s_rel in range(4):
        off = 3 - s_rel

        def shear_body(c, carry):
            for u in range(4):
                m0 = (c * 4 + u) * 16
                row_v[pl.ds(m0, 16)] = line_v[pl.ds(m0 + off, 16)]
            return carry

        jax.lax.fori_loop(0, _LW // 64, shear_body, 0)
        pltpu.sync_copy(row_v, lsw_hbm.at[h, s0 + s_rel])


def _sc_lsw(rel_bias_table):
    table_t = jnp.transpose(rel_bias_table)
    run = functools.partial(
        pl.kernel,
        out_type=jax.ShapeDtypeStruct((_HEADS, 8, _LW), jnp.float32),
        mesh=plsc.VectorSubcoreMesh(
            core_axis_name="c", subcore_axis_name="s",
            num_cores=2, num_subcores=16),
        scratch_types=[
            pltpu.VMEM((_NUM_BUCKETS,), jnp.float32),
            pltpu.VMEM((_LW + _LINE_PAD,), jnp.float32),
            pltpu.VMEM((_LW,), jnp.float32),
        ],
    )(_sc_lsw_kernel)
    return run(table_t)


def _group_adds(w, qk_ref, out_ref):
    for g in range(_BI // 8):
        off = (_BI - 8) - 8 * g
        r = 8 * g
        out_ref[0, 0, r:r + 8, :] = (
            qk_ref[0, 0, r:r + 8, :] + w[:, off:off + _SEQ])


def _bias_add_chain_kernel(table_ref, qk_ref, out_ref, lsw_ref):
    ib = pl.program_id(1)
    n_ib = pl.num_programs(1)

    @pl.when(ib == 0)
    def _():
        sub = jax.lax.broadcasted_iota(jnp.int32, (8, _LW), 0)
        lane = jax.lax.broadcasted_iota(jnp.int32, (8, _LW), 1)
        n = jnp.maximum((_SEQ - _BI) + (_BI - 8) + sub - lane, 0)
        large = 15 + sum((n >= t).astype(jnp.int32) for t in _THRESHOLDS)
        bucket = jnp.where(n < 16, n, jnp.where(n == 16, 0, large))
        w = jnp.full((8, _LW), table_ref[0, 0, 0] * _SCALE, jnp.float32)
        for b in range(1, _MAX_BUCKET + 1):
            w = jnp.where(bucket == b, table_ref[0, 0, b] * _SCALE, w)
        lsw_ref[...] = w

    w = lsw_ref[:, pl.ds(_BI * (n_ib - 1 - ib), _WW)]
    _group_adds(w, qk_ref, out_ref)


def _bias_add_lsw_kernel(lsw_ref, qk_ref, prev_ref, out_ref):
    del prev_ref
    ib = pl.program_id(1)
    n_ib = pl.num_programs(1)
    w = lsw_ref[0, :, pl.ds(_BI * (n_ib - 1 - ib), _WW)]
    _group_adds(w, qk_ref, out_ref)


def kernel(qk_dots, rel_bias_table):
    lsw = _sc_lsw(rel_bias_table)
    n_ib = _SEQ // _BI
    nh1 = _HEADS // 2
    table_t = jnp.transpose(rel_bias_table).reshape(_HEADS, 1, _NUM_BUCKETS)

    part1 = pl.pallas_call(
        _bias_add_chain_kernel,
        grid=(nh1, n_ib),
        in_specs=[
            pl.BlockSpec((1, 1, _NUM_BUCKETS), lambda h, ib: (h, 0, 0)),
            pl.BlockSpec((1, 1, _BI, _SEQ), lambda h, ib: (0, h, ib, 0)),
        ],
        out_specs=pl.BlockSpec((1, 1, _BI, _SEQ), lambda h, ib: (0, h, ib, 0)),
        out_shape=jax.ShapeDtypeStruct(qk_dots.shape, qk_dots.dtype),
        scratch_shapes=[pltpu.VMEM((8, _LW), jnp.float32)],
        compiler_params=pltpu.CompilerParams(
            dimension_semantics=("parallel", "arbitrary")),
    )(table_t, qk_dots)

    return pl.pallas_call(
        _bias_add_lsw_kernel,
        grid=(nh1, n_ib),
        in_specs=[
            pl.BlockSpec((1, 8, _LW), lambda h, ib: (h + nh1, 0, 0)),
            pl.BlockSpec((1, 1, _BI, _SEQ), lambda h, ib: (0, h + nh1, ib, 0)),
            pl.BlockSpec(memory_space=pl.ANY),
        ],
        out_specs=pl.BlockSpec((1, 1, _BI, _SEQ), lambda h, ib: (0, h + nh1, ib, 0)),
        out_shape=jax.ShapeDtypeStruct(qk_dots.shape, qk_dots.dtype),
        input_output_aliases={2: 0},
        compiler_params=pltpu.CompilerParams(
            dimension_semantics=("parallel", "parallel")),
    )(lsw, qk_dots, part1)


# speedup vs baseline: 4.2595x; 1.1326x over previous
import functools

import jax
import jax.numpy as jnp
from jax.experimental import pallas as pl
from jax.experimental.pallas import tpu as pltpu
from jax.experimental.pallas import tpu_sc as plsc

_HEADS = 16
_NUM_BUCKETS = 32
_SCALE = 0.125
_THRESHOLDS = (19, 23, 42, 218)
_MAX_BUCKET = 19

_BI = 512
_SEQ = 2048
_WW = _BI + _SEQ
_LW = 4096
_LINE_PAD = 16


def _sc_lsw_kernel(table_hbm, lsw_hbm, tcol_v, line_v, row_v):
    wid = jax.lax.axis_index("c") * 16 + jax.lax.axis_index("s")
    h = wid // 2
    s0 = 4 * (wid % 2)

    pltpu.sync_copy(table_hbm.at[h], tcol_v)
    t_lo = tcol_v[pl.ds(0, 16)] * _SCALE
    t_hi = tcol_v[pl.ds(16, 16)] * _SCALE

    lane = jax.lax.broadcasted_iota(jnp.int32, (16,), 0)
    base = 2043 + s0
    zero = jnp.zeros((16,), jnp.int32)
    one = jnp.full((16,), 1, jnp.int32)
    k16 = jnp.full((16,), 16, jnp.int32)

    def line_body(c, carry):
        for u in range(4):
            m0 = (c * 4 + u) * 16
            n = jnp.maximum(jnp.broadcast_to(base - m0, (16,)) - lane, zero)
            large = jnp.full((16,), 15, jnp.int32)
            for t in _THRESHOLDS:
                large = large + jnp.where(n >= jnp.full((16,), t, jnp.int32),
                                          one, zero)
            bucket = jnp.where(n < k16, n, jnp.where(n == k16, zero, large))
            v_lo = t_lo.at[jnp.minimum(bucket, 15)].get(
                mode="promise_in_bounds")
            v_hi = t_hi.at[jnp.maximum(bucket - 16, 0)].get(
                mode="promise_in_bounds")
            line_v[pl.ds(m0, 16)] = jnp.where(bucket < k16, v_lo, v_hi)
        return carry

    jax.lax.fori_loop(0, (_LW + _LINE_PAD) // 64, line_body, 0)

    for s_rel in range(4):
        off = 3 - s_rel

        def shear_body(c, carry):
            for u in range(4):
                m0 = (c * 4 + u) * 16
                row_v[pl.ds(m0, 16)] = line_v[pl.ds(m0 + off, 16)]
            return carry

        jax.lax.fori_loop(0, _LW // 64, shear_body, 0)
        pltpu.sync_copy(row_v, lsw_hbm.at[h, s0 + s_rel])


def _sc_lsw(rel_bias_table):
    table_t = jnp.transpose(rel_bias_table)
    run = functools.partial(
        pl.kernel,
        out_type=jax.ShapeDtypeStruct((_HEADS, 8, _LW), jnp.float32),
        mesh=plsc.VectorSubcoreMesh(
            core_axis_name="c", subcore_axis_name="s",
            num_cores=2, num_subcores=16),
        scratch_types=[
            pltpu.VMEM((_NUM_BUCKETS,), jnp.float32),
            pltpu.VMEM((_LW + _LINE_PAD,), jnp.float32),
            pltpu.VMEM((_LW,), jnp.float32),
        ],
    )(_sc_lsw_kernel)
    return run(table_t)


def _group_adds(w, qk_ref, out_ref):
    for g in range(_BI // 8):
        off = (_BI - 8) - 8 * g
        r = 8 * g
        out_ref[0, 0, r:r + 8, :] = (
            qk_ref[0, 0, r:r + 8, :] + w[:, off:off + _SEQ])


def _bias_add_chain_kernel(table_ref, qk_ref, out_ref, lsw_ref):
    ib = pl.program_id(1)
    n_ib = pl.num_programs(1)

    @pl.when(ib == 0)
    def _():
        sub = jax.lax.broadcasted_iota(jnp.int32, (8, _LW), 0)
        lane = jax.lax.broadcasted_iota(jnp.int32, (8, _LW), 1)
        n = jnp.maximum((_SEQ - _BI) + (_BI - 8) + sub - lane, 0)
        large = 15 + sum((n >= t).astype(jnp.int32) for t in _THRESHOLDS)
        bucket = jnp.where(n < 16, n, jnp.where(n == 16, 0, large))
        w = jnp.full((8, _LW), table_ref[0, 0, 0] * _SCALE, jnp.float32)
        for b in range(1, _MAX_BUCKET + 1):
            w = jnp.where(bucket == b, table_ref[0, 0, b] * _SCALE, w)
        lsw_ref[...] = w

    w = lsw_ref[:, pl.ds(_BI * (n_ib - 1 - ib), _WW)]
    _group_adds(w, qk_ref, out_ref)


def _bias_add_lsw_kernel(lsw_ref, qk_ref, prev_ref, out_ref):
    del prev_ref
    ib = pl.program_id(1)
    n_ib = pl.num_programs(1)
    w = lsw_ref[0, :, pl.ds(_BI * (n_ib - 1 - ib), _WW)]
    _group_adds(w, qk_ref, out_ref)


def kernel(qk_dots, rel_bias_table):
    lsw = _sc_lsw(rel_bias_table)
    n_ib = _SEQ // _BI
    nh1 = _HEADS // 2
    table_t = jnp.transpose(rel_bias_table).reshape(_HEADS, 1, _NUM_BUCKETS)

    part1 = pl.pallas_call(
        _bias_add_chain_kernel,
        grid=(nh1, n_ib),
        in_specs=[
            pl.BlockSpec((1, 1, _NUM_BUCKETS), lambda h, ib: (h, 0, 0)),
            pl.BlockSpec((1, 1, _BI, _SEQ), lambda h, ib: (0, h, ib, 0)),
        ],
        out_specs=pl.BlockSpec((1, 1, _BI, _SEQ), lambda h, ib: (0, h, ib, 0)),
        out_shape=jax.ShapeDtypeStruct(qk_dots.shape, qk_dots.dtype),
        scratch_shapes=[pltpu.VMEM((8, _LW), jnp.float32)],
        compiler_params=pltpu.CompilerParams(
            dimension_semantics=("parallel", "arbitrary")),
    )(table_t, qk_dots)

    return pl.pallas_call(
        _bias_add_lsw_kernel,
        grid=(nh1, n_ib),
        in_specs=[
            pl.BlockSpec((1, 8, _LW), lambda h, ib: (h + nh1, 0, 0)),
            pl.BlockSpec((1, 1, _BI, _SEQ), lambda h, ib: (0, h + nh1, ib, 0)),
            pl.BlockSpec(memory_space=pl.ANY),
        ],
        out_specs=pl.BlockSpec((1, 1, _BI, _SEQ), lambda h, ib: (0, h + nh1, ib, 0)),
        out_shape=jax.ShapeDtypeStruct(qk_dots.shape, qk_dots.dtype),
        input_output_aliases={2: 0},
        compiler_params=pltpu.CompilerParams(
            dimension_semantics=("parallel", "parallel")),
    )(lsw, qk_dots, part1)


# speedup vs baseline: 4.3594x; 1.0234x over previous
import functools

import jax
import jax.numpy as jnp
from jax.experimental import pallas as pl
from jax.experimental.pallas import tpu as pltpu
from jax.experimental.pallas import tpu_sc as plsc

_HEADS = 16
_NUM_BUCKETS = 32
_SCALE = 0.125
_THRESHOLDS = (19, 23, 42, 218)
_MAX_BUCKET = 19

_BI = 1024
_SEQ = 2048
_WW = _BI + _SEQ
_LW = 4096
_LINE_PAD = 16


def _sc_lsw_kernel(table_hbm, lsw_hbm, tcol_v, line_v, row_v):
    wid = jax.lax.axis_index("c") * 16 + jax.lax.axis_index("s")
    h = wid // 2
    s0 = 4 * (wid % 2)

    pltpu.sync_copy(table_hbm.at[h], tcol_v)
    t_lo = tcol_v[pl.ds(0, 16)] * _SCALE
    t_hi = tcol_v[pl.ds(16, 16)] * _SCALE

    lane = jax.lax.broadcasted_iota(jnp.int32, (16,), 0)
    base = 2043 + s0
    zero = jnp.zeros((16,), jnp.int32)
    one = jnp.full((16,), 1, jnp.int32)
    k16 = jnp.full((16,), 16, jnp.int32)

    def line_body(c, carry):
        for u in range(4):
            m0 = (c * 4 + u) * 16
            n = jnp.maximum(jnp.broadcast_to(base - m0, (16,)) - lane, zero)
            large = jnp.full((16,), 15, jnp.int32)
            for t in _THRESHOLDS:
                large = large + jnp.where(n >= jnp.full((16,), t, jnp.int32),
                                          one, zero)
            bucket = jnp.where(n < k16, n, jnp.where(n == k16, zero, large))
            v_lo = t_lo.at[jnp.minimum(bucket, 15)].get(
                mode="promise_in_bounds")
            v_hi = t_hi.at[jnp.maximum(bucket - 16, 0)].get(
                mode="promise_in_bounds")
            line_v[pl.ds(m0, 16)] = jnp.where(bucket < k16, v_lo, v_hi)
        return carry

    jax.lax.fori_loop(0, (_LW + _LINE_PAD) // 64, line_body, 0)

    for s_rel in range(4):
        off = 3 - s_rel

        def shear_body(c, carry):
            for u in range(4):
                m0 = (c * 4 + u) * 16
                row_v[pl.ds(m0, 16)] = line_v[pl.ds(m0 + off, 16)]
            return carry

        jax.lax.fori_loop(0, _LW // 64, shear_body, 0)
        pltpu.sync_copy(row_v, lsw_hbm.at[h, s0 + s_rel])


def _sc_lsw(rel_bias_table):
    table_t = jnp.transpose(rel_bias_table)
    run = functools.partial(
        pl.kernel,
        out_type=jax.ShapeDtypeStruct((_HEADS, 8, _LW), jnp.float32),
        mesh=plsc.VectorSubcoreMesh(
            core_axis_name="c", subcore_axis_name="s",
            num_cores=2, num_subcores=16),
        scratch_types=[
            pltpu.VMEM((_NUM_BUCKETS,), jnp.float32),
            pltpu.VMEM((_LW + _LINE_PAD,), jnp.float32),
            pltpu.VMEM((_LW,), jnp.float32),
        ],
    )(_sc_lsw_kernel)
    return run(table_t)


def _group_adds(w, qk_ref, out_ref):
    for g in range(_BI // 8):
        off = (_BI - 8) - 8 * g
        r = 8 * g
        out_ref[0, 0, r:r + 8, :] = (
            qk_ref[0, 0, r:r + 8, :] + w[:, off:off + _SEQ])


def _bias_add_chain_kernel(table_ref, qk_ref, out_ref, lsw_ref):
    ib = pl.program_id(1)
    n_ib = pl.num_programs(1)

    @pl.when(ib == 0)
    def _():
        sub = jax.lax.broadcasted_iota(jnp.int32, (8, _LW), 0)
        lane = jax.lax.broadcasted_iota(jnp.int32, (8, _LW), 1)
        n = jnp.maximum((_SEQ - _BI) + (_BI - 8) + sub - lane, 0)
        large = 15 + sum((n >= t).astype(jnp.int32) for t in _THRESHOLDS)
        bucket = jnp.where(n < 16, n, jnp.where(n == 16, 0, large))
        w = jnp.full((8, _LW), table_ref[0, 0, 0] * _SCALE, jnp.float32)
        for b in range(1, _MAX_BUCKET + 1):
            w = jnp.where(bucket == b, table_ref[0, 0, b] * _SCALE, w)
        lsw_ref[...] = w

    w = lsw_ref[:, pl.ds(_BI * (n_ib - 1 - ib), _WW)]
    _group_adds(w, qk_ref, out_ref)


def _bias_add_lsw_kernel(lsw_ref, qk_ref, prev_ref, out_ref):
    del prev_ref
    ib = pl.program_id(1)
    n_ib = pl.num_programs(1)
    w = lsw_ref[0, :, pl.ds(_BI * (n_ib - 1 - ib), _WW)]
    _group_adds(w, qk_ref, out_ref)


def kernel(qk_dots, rel_bias_table):
    lsw = _sc_lsw(rel_bias_table)
    n_ib = _SEQ // _BI
    nh1 = _HEADS // 2
    table_t = jnp.transpose(rel_bias_table).reshape(_HEADS, 1, _NUM_BUCKETS)

    part1 = pl.pallas_call(
        _bias_add_chain_kernel,
        grid=(nh1, n_ib),
        in_specs=[
            pl.BlockSpec((1, 1, _NUM_BUCKETS), lambda h, ib: (h, 0, 0)),
            pl.BlockSpec((1, 1, _BI, _SEQ), lambda h, ib: (0, h, ib, 0)),
        ],
        out_specs=pl.BlockSpec((1, 1, _BI, _SEQ), lambda h, ib: (0, h, ib, 0)),
        out_shape=jax.ShapeDtypeStruct(qk_dots.shape, qk_dots.dtype),
        scratch_shapes=[pltpu.VMEM((8, _LW), jnp.float32)],
        compiler_params=pltpu.CompilerParams(
            dimension_semantics=("parallel", "arbitrary")),
    )(table_t, qk_dots)

    return pl.pallas_call(
        _bias_add_lsw_kernel,
        grid=(nh1, n_ib),
        in_specs=[
            pl.BlockSpec((1, 8, _LW), lambda h, ib: (h + nh1, 0, 0)),
            pl.BlockSpec((1, 1, _BI, _SEQ), lambda h, ib: (0, h + nh1, ib, 0)),
            pl.BlockSpec(memory_space=pl.ANY),
        ],
        out_specs=pl.BlockSpec((1, 1, _BI, _SEQ), lambda h, ib: (0, h + nh1, ib, 0)),
        out_shape=jax.ShapeDtypeStruct(qk_dots.shape, qk_dots.dtype),
        input_output_aliases={2: 0},
        compiler_params=pltpu.CompilerParams(
            dimension_semantics=("parallel", "parallel")),
    )(lsw, qk_dots, part1)
